# Initial kernel scaffold; baseline (speedup 1.0000x reference)
import jax
import jax.numpy as jnp
from jax.experimental import pallas as pl


def kernel(atomic_embedding, pairlist, f_ij_cutoff, r_ij, W, b):
    raise NotImplementedError("write your pallas kernel here")


# speedup vs baseline: 16.5269x; 16.5269x over previous
import functools

import jax
import jax.numpy as jnp
from jax import lax
from jax.experimental import pallas as pl
from jax.experimental.pallas import tpu as pltpu
from jax.experimental.pallas import tpu_sc as plsc

NC = 2
NS = 16
CH = 128


def _coef_body(f_ref, r_ref, o_ref):
    f = f_ref[0]
    rx = r_ref[0]
    ry = r_ref[1]
    rz = r_ref[2]
    inv = lax.rsqrt(jnp.maximum(rx * rx + ry * ry + rz * rz, 1e-30))
    o_ref[0] = f
    o_ref[1] = f * rx * inv
    o_ref[2] = f * ry * inv
    o_ref[3] = f * rz * inv


def _edge_coefs(f_t, r_t, e_pad, be):
    grid = e_pad // be
    return pl.pallas_call(
        _coef_body,
        grid=(grid,),
        in_specs=[
            pl.BlockSpec((1, be), lambda i: (0, i)),
            pl.BlockSpec((3, be), lambda i: (0, i)),
        ],
        out_specs=pl.BlockSpec((4, be), lambda i: (0, i)),
        out_shape=jax.ShapeDtypeStruct((4, e_pad), jnp.float32),
    )(f_t, r_t)


def _make_sc_kernel(n, d, n_pad, nch):
    rows_per_tile = n_pad // NS
    nz = rows_per_tile // CH

    mesh = plsc.VectorSubcoreMesh(
        core_axis_name="c", subcore_axis_name="s", num_cores=NC, num_subcores=NS
    )

    @functools.partial(
        pl.kernel,
        out_type=[
            jax.ShapeDtypeStruct((4, n_pad, d), jnp.float32),
            jax.ShapeDtypeStruct((NC, n_pad, d), jnp.float32),
        ],
        mesh=mesh,
        scratch_types=[
            pltpu.VMEM((2, CH), jnp.int32),
            pltpu.VMEM((1, CH), jnp.float32),
            pltpu.VMEM((CH, d), jnp.float32),
            pltpu.VMEM_SHARED((n_pad, d), jnp.float32),
            pltpu.SemaphoreType.DMA,
        ],
    )
    def sc_kernel(emb_hbm, meta_hbm, coef_hbm,
                  gall_hbm, cnt_hbm,
                  meta_v, coef_v, rows_v,
                  acc, sem):
        cid = lax.axis_index("c")
        sid = lax.axis_index("s")
        base = sid * rows_per_tile

        def _fill_rows(val):
            def body(r, carry):
                for k in range(d // 16):
                    rows_v[r, pl.ds(k * 16, 16)] = jnp.full(
                        (16,), val, jnp.float32)
                return carry
            lax.fori_loop(0, CH, body, 0)

        def _zero_acc():
            _fill_rows(0.0)
            for z in range(nz):
                pltpu.sync_copy(rows_v, acc.at[pl.ds(base + z * CH, CH)])

        for p in range(2):
            ch = 2 * p + cid

            _zero_acc()
            plsc.subcore_barrier()

            def _chunk(j, carry):
                pltpu.sync_copy(meta_hbm.at[sid, j], meta_v)
                pltpu.sync_copy(coef_hbm.at[ch, sid, j], coef_v)
                pltpu.async_copy(emb_hbm.at[meta_v.at[0]], rows_v, sem).wait()

                def _scale(eg, c2):
                    cvec = coef_v[0, pl.ds(eg * 16, 16)]
                    ebase = eg * 16
                    for lane in range(16):
                        s = cvec[lane]
                        for k in range(d // 16):
                            sl = pl.ds(k * 16, 16)
                            rows_v[ebase + lane, sl] = rows_v[ebase + lane, sl] * s
                    return c2
                lax.fori_loop(0, CH // 16, _scale, 0)

                pltpu.sync_copy(rows_v, acc.at[meta_v.at[1]], add=True)
                return carry
            lax.fori_loop(0, nch, _chunk, 0)
            plsc.subcore_barrier()

            for z in range(nz):
                rows = pl.ds(base + z * CH, CH)
                pltpu.sync_copy(acc.at[rows], gall_hbm.at[ch, rows])
            plsc.subcore_barrier()

        _zero_acc()
        plsc.subcore_barrier()
        _fill_rows(1.0)

        def _cchunk(j, carry):
            pltpu.sync_copy(meta_hbm.at[sid, j], meta_v)
            pltpu.sync_copy(rows_v, acc.at[meta_v.at[1]], add=True)
            return carry
        half = nch // 2
        lax.fori_loop(cid * half, cid * half + half, _cchunk, 0)
        plsc.subcore_barrier()
        for z in range(nz):
            rows = pl.ds(base + z * CH, CH)
            pltpu.sync_copy(acc.at[rows], cnt_hbm.at[cid, rows])

    return sc_kernel


def _combine_body(g_ref, cnt_ref, wt_ref, b_ref, o_ref):
    wt = wt_ref[...]
    bias = (cnt_ref[0, :, 0:1] + cnt_ref[1, :, 0:1]) * b_ref[...]
    s0 = jnp.dot(g_ref[1], wt, preferred_element_type=jnp.float32) + bias
    s1 = jnp.dot(g_ref[2], wt, preferred_element_type=jnp.float32) + bias
    s2 = jnp.dot(g_ref[3], wt, preferred_element_type=jnp.float32) + bias
    o_ref[:, 0:128] = jnp.sqrt(s0 * s0 + s1 * s1 + s2 * s2)
    o_ref[:, 128:256] = g_ref[0]


def _combine(gall, cnt, wt, b2, n, d, bn):
    grid = n // bn
    return pl.pallas_call(
        _combine_body,
        grid=(grid,),
        in_specs=[
            pl.BlockSpec((4, bn, d), lambda i: (0, i, 0)),
            pl.BlockSpec((NC, bn, d), lambda i: (0, i, 0)),
            pl.BlockSpec((d, d), lambda i: (0, 0)),
            pl.BlockSpec((1, d), lambda i: (0, 0)),
        ],
        out_specs=pl.BlockSpec((bn, 2 * d), lambda i: (i, 0)),
        out_shape=jax.ShapeDtypeStruct((n, 2 * d), jnp.float32),
    )(gall, cnt, wt, b2)


def kernel(atomic_embedding, pairlist, f_ij_cutoff, r_ij, W, b):
    n, d = atomic_embedding.shape
    e = pairlist.shape[1]
    ept_unit = NS * CH * 2
    e_pad = ((e + ept_unit - 1) // ept_unit) * ept_unit
    nch = e_pad // (NS * CH)
    n_pad = ((n + NS * CH - 1) // (NS * CH)) * (NS * CH)

    idx_i = pairlist[0]
    idx_j = pairlist[1]
    pad_e = e_pad - e

    f_t = jnp.pad(f_ij_cutoff.T, ((0, 0), (0, pad_e)))
    r_t = jnp.pad(r_ij.T, ((0, 0), (0, pad_e)))
    idxj_r = jnp.pad(idx_j, (0, pad_e)).reshape(NS, nch, 1, CH)
    idxi_r = jnp.pad(idx_i, (0, pad_e),
                     constant_values=n).reshape(NS, nch, 1, CH)
    meta = jnp.concatenate([idxj_r, idxi_r], axis=2)

    coef = _edge_coefs(f_t, r_t, e_pad, ept_unit)
    coef_r = coef.reshape(4, NS, nch, 1, CH)

    sc_kernel = _make_sc_kernel(n, d, n_pad, nch)
    gall, cnt = sc_kernel(atomic_embedding, meta, coef_r)

    wt = W.T
    b2 = b.reshape(1, d)
    return _combine(gall, cnt, wt, b2, n, d, 400)


# speedup vs baseline: 18.9273x; 1.1452x over previous
import functools

import jax
import jax.numpy as jnp
from jax import lax
from jax.experimental import pallas as pl
from jax.experimental.pallas import tpu as pltpu
from jax.experimental.pallas import tpu_sc as plsc

NC = 2
NS = 16
CH = 128


def _coef_body(f_ref, r_ref, o_ref):
    f = f_ref[0]
    rx = r_ref[0]
    ry = r_ref[1]
    rz = r_ref[2]
    inv = lax.rsqrt(jnp.maximum(rx * rx + ry * ry + rz * rz, 1e-30))
    o_ref[0] = f
    o_ref[1] = f * rx * inv
    o_ref[2] = f * ry * inv
    o_ref[3] = f * rz * inv


def _edge_coefs(f_t, r_t, e_pad, be):
    grid = e_pad // be
    return pl.pallas_call(
        _coef_body,
        grid=(grid,),
        in_specs=[
            pl.BlockSpec((1, be), lambda i: (0, i)),
            pl.BlockSpec((3, be), lambda i: (0, i)),
        ],
        out_specs=pl.BlockSpec((4, be), lambda i: (0, i)),
        out_shape=jax.ShapeDtypeStruct((4, e_pad), jnp.float32),
    )(f_t, r_t)


def _make_sc_kernel(n, d, n_pad, nch):
    rows_per_tile = n_pad // NS
    nz = rows_per_tile // CH

    mesh = plsc.VectorSubcoreMesh(
        core_axis_name="c", subcore_axis_name="s", num_cores=NC, num_subcores=NS
    )

    @functools.partial(
        pl.kernel,
        out_type=[
            jax.ShapeDtypeStruct((4, n_pad, d), jnp.float32),
            jax.ShapeDtypeStruct((NC, n_pad, d), jnp.float32),
        ],
        mesh=mesh,
        scratch_types=[
            pltpu.VMEM((2, 2, CH), jnp.int32),
            pltpu.VMEM((2, 1, CH), jnp.float32),
            pltpu.VMEM((2, CH, d), jnp.float32),
            pltpu.VMEM_SHARED((n_pad, d), jnp.float32),
            pltpu.SemaphoreType.DMA,
            pltpu.SemaphoreType.DMA,
            pltpu.SemaphoreType.DMA,
            pltpu.SemaphoreType.DMA,
        ],
    )
    def sc_kernel(emb_hbm, meta_hbm, coef_hbm,
                  gall_hbm, cnt_hbm,
                  meta_v, coef_v, rows_v,
                  acc, gs0, gs1, ss0, ss1):
        cid = lax.axis_index("c")
        sid = lax.axis_index("s")
        base = sid * rows_per_tile
        gsems = (gs0, gs1)
        ssems = (ss0, ss1)

        def _fill_rows0(val):
            def body(r, carry):
                for k in range(d // 16):
                    rows_v[0, r, pl.ds(k * 16, 16)] = jnp.full(
                        (16,), val, jnp.float32)
                return carry
            lax.fori_loop(0, CH, body, 0)

        def _zero_acc():
            _fill_rows0(0.0)
            for z in range(nz):
                pltpu.sync_copy(rows_v.at[0], acc.at[pl.ds(base + z * CH, CH)])

        def _stage(ch, j, buf):
            pltpu.sync_copy(meta_hbm.at[sid, j], meta_v.at[buf])
            pltpu.sync_copy(coef_hbm.at[ch, sid, j], coef_v.at[buf])

        def _gather(buf):
            pltpu.async_copy(
                emb_hbm.at[meta_v.at[buf, 0]], rows_v.at[buf], gsems[buf])

        def _wait_gather(buf):
            pltpu.make_async_copy(
                emb_hbm.at[meta_v.at[buf, 0]], rows_v.at[buf],
                gsems[buf]).wait()

        def _scatter(buf):
            pltpu.async_copy(
                rows_v.at[buf], acc.at[meta_v.at[buf, 1]], ssems[buf],
                add=True)

        def _wait_scatter(buf):
            pltpu.make_async_copy(
                rows_v.at[buf], acc.at[meta_v.at[buf, 1]], ssems[buf]).wait()

        def _scale(buf):
            def body(eg, c2):
                cvec = coef_v[buf, 0, pl.ds(eg * 16, 16)]
                ebase = eg * 16
                for lane in range(16):
                    sv = jnp.broadcast_to(cvec[lane], (16,))
                    for k in range(d // 16):
                        sl = pl.ds(k * 16, 16)
                        rows_v[buf, ebase + lane, sl] = (
                            rows_v[buf, ebase + lane, sl] * sv)
                return c2
            lax.fori_loop(0, CH // 16, body, 0)

        npair = nch // 2

        for p in range(2):
            ch = 2 * p + cid

            _zero_acc()
            plsc.subcore_barrier()

            _stage(ch, 0, 0)
            _gather(0)
            _stage(ch, 1, 1)
            _gather(1)

            def _pair(jj, carry):
                _wait_gather(0)
                _scale(0)
                _scatter(0)
                _wait_gather(1)
                _scale(1)
                _scatter(1)

                @pl.when(jj < npair - 1)
                def _prefetch():
                    _wait_scatter(0)
                    _stage(ch, 2 * jj + 2, 0)
                    _gather(0)
                    _wait_scatter(1)
                    _stage(ch, 2 * jj + 3, 1)
                    _gather(1)
                return carry
            lax.fori_loop(0, npair, _pair, 0)
            _wait_scatter(0)
            _wait_scatter(1)
            plsc.subcore_barrier()

            for z in range(nz):
                rows = pl.ds(base + z * CH, CH)
                pltpu.sync_copy(acc.at[rows], gall_hbm.at[ch, rows])
            plsc.subcore_barrier()

        _zero_acc()
        plsc.subcore_barrier()
        _fill_rows0(1.0)

        def _cchunk(j, carry):
            pltpu.sync_copy(meta_hbm.at[sid, j], meta_v.at[0])
            pltpu.sync_copy(rows_v.at[0], acc.at[meta_v.at[0, 1]], add=True)
            return carry
        half = nch // 2
        lax.fori_loop(cid * half, cid * half + half, _cchunk, 0)
        plsc.subcore_barrier()
        for z in range(nz):
            rows = pl.ds(base + z * CH, CH)
            pltpu.sync_copy(acc.at[rows], cnt_hbm.at[cid, rows])

    return sc_kernel


def _combine_body(g_ref, cnt_ref, wt_ref, b_ref, o_ref):
    wt = wt_ref[...]
    bias = (cnt_ref[0, :, 0:1] + cnt_ref[1, :, 0:1]) * b_ref[...]
    s0 = jnp.dot(g_ref[1], wt, preferred_element_type=jnp.float32) + bias
    s1 = jnp.dot(g_ref[2], wt, preferred_element_type=jnp.float32) + bias
    s2 = jnp.dot(g_ref[3], wt, preferred_element_type=jnp.float32) + bias
    o_ref[:, 0:128] = jnp.sqrt(s0 * s0 + s1 * s1 + s2 * s2)
    o_ref[:, 128:256] = g_ref[0]


def _combine(gall, cnt, wt, b2, n, d, bn):
    grid = n // bn
    return pl.pallas_call(
        _combine_body,
        grid=(grid,),
        in_specs=[
            pl.BlockSpec((4, bn, d), lambda i: (0, i, 0)),
            pl.BlockSpec((NC, bn, d), lambda i: (0, i, 0)),
            pl.BlockSpec((d, d), lambda i: (0, 0)),
            pl.BlockSpec((1, d), lambda i: (0, 0)),
        ],
        out_specs=pl.BlockSpec((bn, 2 * d), lambda i: (i, 0)),
        out_shape=jax.ShapeDtypeStruct((n, 2 * d), jnp.float32),
    )(gall, cnt, wt, b2)


def kernel(atomic_embedding, pairlist, f_ij_cutoff, r_ij, W, b):
    n, d = atomic_embedding.shape
    e = pairlist.shape[1]
    ept_unit = NS * CH * 2
    e_pad = ((e + ept_unit - 1) // ept_unit) * ept_unit
    nch = e_pad // (NS * CH)
    n_pad = ((n + NS * CH - 1) // (NS * CH)) * (NS * CH)

    idx_i = pairlist[0]
    idx_j = pairlist[1]
    pad_e = e_pad - e

    f_t = jnp.pad(f_ij_cutoff.T, ((0, 0), (0, pad_e)))
    r_t = jnp.pad(r_ij.T, ((0, 0), (0, pad_e)))
    idxj_r = jnp.pad(idx_j, (0, pad_e)).reshape(NS, nch, 1, CH)
    idxi_r = jnp.pad(idx_i, (0, pad_e),
                     constant_values=n).reshape(NS, nch, 1, CH)
    meta = jnp.concatenate([idxj_r, idxi_r], axis=2)

    coef = _edge_coefs(f_t, r_t, e_pad, ept_unit)
    coef_r = coef.reshape(4, NS, nch, 1, CH)

    sc_kernel = _make_sc_kernel(n, d, n_pad, nch)
    gall, cnt = sc_kernel(atomic_embedding, meta, coef_r)

    wt = W.T
    b2 = b.reshape(1, d)
    return _combine(gall, cnt, wt, b2, n, d, 400)
